# baseline (speedup 1.0000x reference)
import jax
import jax.numpy as jnp
from jax.experimental import pallas as pl
from jax.experimental.pallas import tpu as pltpu

_B, _L, _D = 16384, 200, 32
_BB = 128


def _body(x_ref, m_ref, o_ref):
    x = x_ref[...]
    m = m_ref[...].astype(jnp.float32)
    mb = jax.lax.broadcast_in_dim(m, x.shape, (0, 1))
    o_ref[...] = jnp.sum(x * mb, axis=1)


def kernel(inputs, mask):
    return pl.pallas_call(
        _body,
        grid=(_B // _BB,),
        in_specs=[
            pl.BlockSpec((_BB, _L, _D), lambda i: (i, 0, 0)),
            pl.BlockSpec((_BB, _L), lambda i: (i, 0)),
        ],
        out_specs=pl.BlockSpec((_BB, _D), lambda i: (i, 0)),
        out_shape=jax.ShapeDtypeStruct((_B, _D), jnp.float32),
        compiler_params=pltpu.CompilerParams(
            dimension_semantics=("arbitrary",),
        ),
    )(inputs, mask)


# speedup vs baseline: 1.5631x; 1.5631x over previous
import jax
import jax.numpy as jnp
from jax.experimental import pallas as pl
from jax.experimental.pallas import tpu as pltpu

_B, _L, _D = 16384, 200, 32
_G = 50
_BB = 256


def _body(x_ref, m_ref, o_ref):
    y = x_ref[...]
    m = m_ref[...].astype(jnp.float32)
    acc = jnp.zeros((y.shape[0], _D), jnp.float32)
    for k in range(4):
        mk = jax.lax.broadcast_in_dim(
            m[:, :, k], (y.shape[0], _G, _D), (0, 1))
        acc = acc + jnp.sum(y[:, :, k * _D:(k + 1) * _D] * mk, axis=1)
    o_ref[...] = acc


def kernel(inputs, mask):
    x = jnp.reshape(inputs, (_B, _G, 128))
    m = jnp.reshape(mask, (_B, _G, 4))
    return pl.pallas_call(
        _body,
        grid=(_B // _BB,),
        in_specs=[
            pl.BlockSpec((_BB, _G, 128), lambda i: (i, 0, 0)),
            pl.BlockSpec((_BB, _G, 4), lambda i: (i, 0, 0)),
        ],
        out_specs=pl.BlockSpec((_BB, _D), lambda i: (i, 0)),
        out_shape=jax.ShapeDtypeStruct((_B, _D), jnp.float32),
        compiler_params=pltpu.CompilerParams(
            dimension_semantics=("arbitrary",),
        ),
    )(x, m)
